# baseline (speedup 1.0000x reference)
import functools

import jax
import jax.numpy as jnp
from jax import lax
from jax.experimental import pallas as pl
from jax.experimental.pallas import tpu as pltpu
from jax.experimental.pallas import tpu_sc as plsc

N = 10000
E = 320000
D_IN = 128
D_H = 256

NC = 2
NS = 16
NW = NC * NS

N_PAD = 10240
ROWS_PER_TILE = N_PAD // NS

CHUNK = 32
NB = 8
KMAX = 320
E_PAD = KMAX * NW * CHUNK


def _sc_mesh():
  return plsc.VectorSubcoreMesh(
      core_axis_name="c", subcore_axis_name="s",
      num_cores=NC, num_subcores=NS)


def _edge_pipeline(nworkers, wid, table_at, src_h, dst_h, acc_sh, deg_sh,
                   src_v, dst_v, rows_v, ones_v, sem_i, sem_g, sem_s,
                   with_deg):
  nchunks = E_PAD // (nworkers * CHUNK)

  def idx_start(j, p, s):
    off = ((j * NB + s) * nworkers + wid) * CHUNK
    pltpu.async_copy(src_h.at[pl.ds(off, CHUNK)], src_v.at[p, s], sem_i[s])
    pltpu.async_copy(dst_h.at[pl.ds(off, CHUNK)], dst_v.at[p, s], sem_i[s])

  def idx_wait(p, s):
    pltpu.make_async_copy(
        src_h.at[pl.ds(0, CHUNK)], src_v.at[p, s], sem_i[s]).wait()
    pltpu.make_async_copy(
        dst_h.at[pl.ds(0, CHUNK)], dst_v.at[p, s], sem_i[s]).wait()

  def gath_start(p, s):
    pltpu.async_copy(table_at(src_v.at[p, s]), rows_v.at[s], sem_g[s])

  def gath_wait(p, s):
    pltpu.make_async_copy(
        table_at(src_v.at[p, s]), rows_v.at[s], sem_g[s]).wait()

  def scat_start(p, s):
    pltpu.async_copy(rows_v.at[s], acc_sh.at[dst_v.at[p, s]], sem_s[s],
                     add=True)
    if with_deg:
      pltpu.async_copy(ones_v, deg_sh.at[dst_v.at[p, s]], sem_s[s],
                       add=True)

  def scat_wait(p, s):
    pltpu.make_async_copy(
        rows_v.at[s], acc_sh.at[dst_v.at[p, s]], sem_s[s]).wait()
    if with_deg:
      pltpu.make_async_copy(
          ones_v, deg_sh.at[dst_v.at[p, s]], sem_s[s]).wait()

  nj = nchunks // NB
  assert nj % 2 == 0
  for s in range(NB):
    idx_start(0, 0, s)
  for s in range(NB):
    idx_wait(0, s)
    gath_start(0, s)

  def block(j, p):
    for s in range(NB):
      gath_wait(p, s)
      scat_start(p, s)

      @pl.when(j + 1 < nj)
      def _():
        idx_start(j + 1, 1 - p, s)
    for s in range(NB):

      @pl.when(j + 1 < nj)
      def _():
        scat_wait(p, s)
        idx_wait(1 - p, s)
        gath_start(1 - p, s)

  def step(i, _):
    block(2 * i, 0)
    block(2 * i + 1, 1)
    return None

  lax.fori_loop(0, nj // 2, step, None)
  for s in range(NB):
    scat_wait(1, s)


def _sc_scratch():
  return [
      pltpu.VMEM((2, NB, CHUNK), jnp.int32),
      pltpu.VMEM((2, NB, CHUNK), jnp.int32),
      pltpu.VMEM((NB, CHUNK, D_IN), jnp.float32),
      pltpu.VMEM((CHUNK,), jnp.float32),
      pltpu.VMEM_SHARED((N_PAD, D_IN), jnp.float32),
      pltpu.VMEM_SHARED((N_PAD,), jnp.float32),
      [pltpu.SemaphoreType.DMA] * NB,
      [pltpu.SemaphoreType.DMA] * NB,
      [pltpu.SemaphoreType.DMA] * NB,
  ]


@jax.jit
def _sc_agg_layer1(table, src, dst, zrows, zvec):
  out_type = [
      jax.ShapeDtypeStruct((NC, N_PAD, D_IN), jnp.float32),
      jax.ShapeDtypeStruct((NC, N_PAD), jnp.float32),
  ]

  def body(table_h, src_h, dst_h, zrows_h, zvec_h, acc_out, deg_out,
           src_v, dst_v, rows_v, ones_v, acc_sh, deg_sh,
           sem_i, sem_g, sem_s):
    cid = lax.axis_index("c")
    sid = lax.axis_index("s")
    row0 = sid * ROWS_PER_TILE
    pltpu.sync_copy(zrows_h, acc_sh.at[pl.ds(row0, ROWS_PER_TILE)])
    pltpu.sync_copy(zvec_h, deg_sh.at[pl.ds(row0, ROWS_PER_TILE)])
    for i in range(CHUNK // 16):
      ones_v[pl.ds(i * 16, 16)] = jnp.full((16,), 1.0, jnp.float32)
    plsc.subcore_barrier()

    _edge_pipeline(NW, sid * NC + cid, lambda idx: table_h.at[idx],
                   src_h, dst_h, acc_sh, deg_sh,
                   src_v, dst_v, rows_v, ones_v, sem_i, sem_g, sem_s,
                   with_deg=True)
    plsc.subcore_barrier()

    pltpu.sync_copy(acc_sh.at[pl.ds(row0, ROWS_PER_TILE)],
                    acc_out.at[cid, pl.ds(row0, ROWS_PER_TILE)])
    pltpu.sync_copy(deg_sh.at[pl.ds(row0, ROWS_PER_TILE)],
                    deg_out.at[cid, pl.ds(row0, ROWS_PER_TILE)])

  fn = pl.kernel(body, out_type=out_type, mesh=_sc_mesh(),
                 scratch_types=_sc_scratch())
  return fn(table, src, dst, zrows, zvec)


@jax.jit
def _sc_agg_layer2(h1, src, dst, zrows):
  out_type = [jax.ShapeDtypeStruct((NC, N_PAD, D_IN), jnp.float32)]

  def body(h1_h, src_h, dst_h, zrows_h, y_out,
           src_v, dst_v, rows_v, ones_v, acc_sh, deg_sh,
           sem_i, sem_g, sem_s):
    cid = lax.axis_index("c")
    sid = lax.axis_index("s")
    row0 = sid * ROWS_PER_TILE
    pltpu.sync_copy(zrows_h, acc_sh.at[pl.ds(row0, ROWS_PER_TILE)])
    plsc.subcore_barrier()

    _edge_pipeline(NS, sid, lambda idx: h1_h.at[cid].at[idx],
                   src_h, dst_h, acc_sh, deg_sh,
                   src_v, dst_v, rows_v, ones_v, sem_i, sem_g, sem_s,
                   with_deg=False)
    plsc.subcore_barrier()

    pltpu.sync_copy(acc_sh.at[pl.ds(row0, ROWS_PER_TILE)],
                    y_out.at[cid, pl.ds(row0, ROWS_PER_TILE)])

  fn = pl.kernel(body, out_type=out_type, mesh=_sc_mesh(),
                 scratch_types=_sc_scratch())
  return fn(h1, src, dst, zrows)[0]


RBLK = 2000
NBLKS = N // RBLK
RBLK1 = 1280
NBLKS1 = N_PAD // RBLK1


def _bf(v):
  return v.astype(jnp.bfloat16)


def _tc1_body(acc, x, deg3, w0, b0, h1):
  deg = deg3[0] + deg3[1] + 1.0
  agg = _bf((acc[0] + acc[1] + x[...]) / deg)
  h1[0] = jnp.maximum(
      jnp.dot(agg, _bf(w0[0]), preferred_element_type=jnp.float32) + b0[0],
      0.0)
  h1[1] = jnp.maximum(
      jnp.dot(agg, _bf(w0[1]), preferred_element_type=jnp.float32) + b0[1],
      0.0)


@jax.jit
def _tc_layer1(acc0, x, deg3, w0s, b0s):
  return pl.pallas_call(
      _tc1_body,
      grid=(NBLKS1,),
      in_specs=[pl.BlockSpec((NC, RBLK1, D_IN), lambda i: (0, i, 0)),
                pl.BlockSpec((RBLK1, D_IN), lambda i: (i, 0)),
                pl.BlockSpec((NC, RBLK1, 1), lambda i: (0, i, 0)),
                pl.BlockSpec((NC, D_IN, D_IN), lambda i: (0, 0, 0)),
                pl.BlockSpec((NC, 1, D_IN), lambda i: (0, 0, 0))],
      out_specs=[pl.BlockSpec((NC, RBLK1, D_IN), lambda i: (0, i, 0))],
      out_shape=[jax.ShapeDtypeStruct((NC, N_PAD, D_IN), jnp.float32)],
  )(acc0, x, deg3, w0s, b0s)[0]


def _tc2_body(y, h1, deg3, orig, al,
              w1, b1, wa1, ba1, wa2, ba2, out):
  deg = deg3[0] + deg3[1] + 1.0
  agga = _bf((y[0] + h1[0]) / deg)
  aggb = _bf((y[1] + h1[1]) / deg)
  h2 = jnp.dot(agga, _bf(w1[...][:D_IN]),
               preferred_element_type=jnp.float32)
  h2 = h2 + jnp.dot(aggb, _bf(w1[...][D_IN:]),
                    preferred_element_type=jnp.float32)
  h2 = _bf(jnp.maximum(h2 + b1[...], 0.0))
  a1 = _bf(jnp.maximum(
      jnp.dot(h2, _bf(wa1[...]), preferred_element_type=jnp.float32)
      + ba1[...], 0.0))
  adj = jnp.dot(a1, _bf(wa2[...]), preferred_element_type=jnp.float32) \
      + ba2[...]
  alpha = 1.0 / (1.0 + jnp.exp(-al[...]))
  out[...] = alpha * orig[...] + (1.0 - alpha) * adj


@jax.jit
def _tc_layer2(y, h1, deg3, orig, al, w1, b1, wa1, ba1, wa2, ba2):
  full = lambda r, c: pl.BlockSpec((r, c), lambda i: (0, 0))
  return pl.pallas_call(
      _tc2_body,
      grid=(NBLKS,),
      in_specs=[pl.BlockSpec((NC, RBLK, D_IN), lambda i: (0, i, 0)),
                pl.BlockSpec((NC, RBLK, D_IN), lambda i: (0, i, 0)),
                pl.BlockSpec((NC, RBLK, 1), lambda i: (0, i, 0)),
                pl.BlockSpec((RBLK, 1), lambda i: (i, 0)), full(1, 1),
                full(D_H, D_H), full(1, D_H),
                full(D_H, D_IN), full(1, D_IN),
                full(D_IN, 1), full(1, 1)],
      out_specs=[pl.BlockSpec((RBLK, 1), lambda i: (i, 0))],
      out_shape=[jax.ShapeDtypeStruct((N, 1), jnp.float32)],
  )(y, h1, deg3, orig, al, w1, b1, wa1, ba1, wa2, ba2)[0]


def kernel(x, edge_index, original_scores, W0, b0, W1, b1, Wa1, ba1, Wa2,
           ba2, alpha_logit):
  extra = jnp.arange(E_PAD - E, dtype=jnp.int32)
  src = jnp.concatenate([edge_index[0], extra % N])
  dst = jnp.concatenate([edge_index[1], N + extra % (N_PAD - N)])
  zrows = jnp.zeros((ROWS_PER_TILE, D_IN), jnp.float32)
  zvec = jnp.zeros((ROWS_PER_TILE,), jnp.float32)
  w0s = jnp.stack([W0[:, :D_IN], W0[:, D_IN:]])
  b0s = jnp.stack([b0[None, :D_IN], b0[None, D_IN:]])

  acc0, deg = _sc_agg_layer1(x, src, dst, zrows, zvec)
  deg3 = deg[:, :, None]

  xp = jnp.concatenate([x, jnp.zeros((N_PAD - N, D_IN), jnp.float32)])
  h1 = _tc_layer1(acc0, xp, deg3, w0s, b0s)
  y = _sc_agg_layer2(h1, src, dst, zrows)

  refined = _tc_layer2(y, h1, deg3, original_scores[:, None],
                       alpha_logit[None, None],
                       W1, b1[None, :], Wa1, ba1[None, :], Wa2,
                       ba2.reshape(1, 1))
  return refined[:, 0]


# speedup vs baseline: 1.0090x; 1.0090x over previous
import functools

import jax
import jax.numpy as jnp
from jax import lax
from jax.experimental import pallas as pl
from jax.experimental.pallas import tpu as pltpu
from jax.experimental.pallas import tpu_sc as plsc

N = 10000
E = 320000
D_IN = 128
D_H = 256

NC = 2
NS = 16
NW = NC * NS

N_PAD = 10240
ROWS_PER_TILE = N_PAD // NS

CHUNK = 32
NB = 8
KMAX = 320
E_PAD = KMAX * NW * CHUNK


def _sc_mesh():
  return plsc.VectorSubcoreMesh(
      core_axis_name="c", subcore_axis_name="s",
      num_cores=NC, num_subcores=NS)


def _edge_pipeline(nworkers, wid, table_at, src_h, dst_h, acc_sh, deg_sh,
                   src_v, dst_v, rows_v, ones_v, sem_i, sem_g, sem_s,
                   with_deg):
  nchunks = E_PAD // (nworkers * CHUNK)

  def idx_start(j, p, s):
    off = ((j * NB + s) * nworkers + wid) * CHUNK
    pltpu.async_copy(src_h.at[pl.ds(off, CHUNK)], src_v.at[p, s], sem_i[s])
    pltpu.async_copy(dst_h.at[pl.ds(off, CHUNK)], dst_v.at[p, s], sem_i[s])

  def idx_wait(p, s):
    pltpu.make_async_copy(
        src_h.at[pl.ds(0, CHUNK)], src_v.at[p, s], sem_i[s]).wait()
    pltpu.make_async_copy(
        dst_h.at[pl.ds(0, CHUNK)], dst_v.at[p, s], sem_i[s]).wait()

  def gath_start(p, s):
    pltpu.async_copy(table_at(src_v.at[p, s]), rows_v.at[s], sem_g[s])

  def gath_wait(p, s):
    pltpu.make_async_copy(
        table_at(src_v.at[p, s]), rows_v.at[s], sem_g[s]).wait()

  def scat_start(p, s):
    pltpu.async_copy(rows_v.at[s], acc_sh.at[dst_v.at[p, s]], sem_s[s],
                     add=True)
    if with_deg:
      pltpu.async_copy(ones_v, deg_sh.at[dst_v.at[p, s]], sem_s[s],
                       add=True)

  def scat_wait(p, s):
    pltpu.make_async_copy(
        rows_v.at[s], acc_sh.at[dst_v.at[p, s]], sem_s[s]).wait()
    if with_deg:
      pltpu.make_async_copy(
          ones_v, deg_sh.at[dst_v.at[p, s]], sem_s[s]).wait()

  nj = nchunks // NB
  assert nj % 2 == 0
  for s in range(NB):
    idx_start(0, 0, s)
  for s in range(NB):
    idx_wait(0, s)
    gath_start(0, s)

  def block(j, p):
    for s in range(NB):
      gath_wait(p, s)
      scat_start(p, s)

      @pl.when(j + 1 < nj)
      def _():
        idx_start(j + 1, 1 - p, s)
    for s in range(NB):

      @pl.when(j + 1 < nj)
      def _():
        scat_wait(p, s)
        idx_wait(1 - p, s)
        gath_start(1 - p, s)

  def step(i, _):
    block(2 * i, 0)
    block(2 * i + 1, 1)
    return None

  lax.fori_loop(0, nj // 2, step, None)
  for s in range(NB):
    scat_wait(1, s)


def _sc_scratch():
  return [
      pltpu.VMEM((2, NB, CHUNK), jnp.int32),
      pltpu.VMEM((2, NB, CHUNK), jnp.int32),
      pltpu.VMEM((NB, CHUNK, D_IN), jnp.float32),
      pltpu.VMEM((CHUNK,), jnp.float32),
      pltpu.VMEM_SHARED((N_PAD, D_IN), jnp.float32),
      pltpu.VMEM_SHARED((N_PAD,), jnp.float32),
      [pltpu.SemaphoreType.DMA] * NB,
      [pltpu.SemaphoreType.DMA] * NB,
      [pltpu.SemaphoreType.DMA] * NB,
  ]


@jax.jit
def _sc_agg_layer1(table, src, dst, zrows, zvec):
  out_type = [
      jax.ShapeDtypeStruct((NC, N_PAD, D_IN), jnp.float32),
      jax.ShapeDtypeStruct((NC, N_PAD), jnp.float32),
  ]

  def body(table_h, src_h, dst_h, zrows_h, zvec_h, acc_out, deg_out,
           src_v, dst_v, rows_v, ones_v, acc_sh, deg_sh,
           sem_i, sem_g, sem_s):
    cid = lax.axis_index("c")
    sid = lax.axis_index("s")
    row0 = sid * ROWS_PER_TILE
    pltpu.sync_copy(zrows_h, acc_sh.at[pl.ds(row0, ROWS_PER_TILE)])
    pltpu.sync_copy(zvec_h, deg_sh.at[pl.ds(row0, ROWS_PER_TILE)])
    for i in range(CHUNK // 16):
      ones_v[pl.ds(i * 16, 16)] = jnp.full((16,), 1.0, jnp.float32)
    plsc.subcore_barrier()

    _edge_pipeline(NW, sid * NC + cid, lambda idx: table_h.at[idx],
                   src_h, dst_h, acc_sh, deg_sh,
                   src_v, dst_v, rows_v, ones_v, sem_i, sem_g, sem_s,
                   with_deg=True)
    plsc.subcore_barrier()

    pltpu.sync_copy(acc_sh.at[pl.ds(row0, ROWS_PER_TILE)],
                    acc_out.at[cid, pl.ds(row0, ROWS_PER_TILE)])
    pltpu.sync_copy(deg_sh.at[pl.ds(row0, ROWS_PER_TILE)],
                    deg_out.at[cid, pl.ds(row0, ROWS_PER_TILE)])

  fn = pl.kernel(body, out_type=out_type, mesh=_sc_mesh(),
                 scratch_types=_sc_scratch())
  return fn(table, src, dst, zrows, zvec)


@jax.jit
def _sc_agg_layer2(h1, src, dst, zrows):
  out_type = [jax.ShapeDtypeStruct((NC, N_PAD, D_IN), jnp.float32)]

  def body(h1_h, src_h, dst_h, zrows_h, y_out,
           src_v, dst_v, rows_v, ones_v, acc_sh, deg_sh,
           sem_i, sem_g, sem_s):
    cid = lax.axis_index("c")
    sid = lax.axis_index("s")
    row0 = sid * ROWS_PER_TILE
    pltpu.sync_copy(zrows_h, acc_sh.at[pl.ds(row0, ROWS_PER_TILE)])
    plsc.subcore_barrier()

    _edge_pipeline(NS, sid, lambda idx: h1_h.at[cid].at[idx],
                   src_h, dst_h, acc_sh, deg_sh,
                   src_v, dst_v, rows_v, ones_v, sem_i, sem_g, sem_s,
                   with_deg=False)
    plsc.subcore_barrier()

    pltpu.sync_copy(acc_sh.at[pl.ds(row0, ROWS_PER_TILE)],
                    y_out.at[cid, pl.ds(row0, ROWS_PER_TILE)])

  fn = pl.kernel(body, out_type=out_type, mesh=_sc_mesh(),
                 scratch_types=_sc_scratch())
  return fn(h1, src, dst, zrows)[0]


RBLK = 2000
NBLKS = N // RBLK


def _tc1_body(acc, x, deg3, w0, b0, h1):
  deg = deg3[0] + deg3[1] + 1.0
  agg = (acc[0] + acc[1] + x[...]) / deg
  h1[0] = jnp.maximum(
      jnp.dot(agg, w0[0], preferred_element_type=jnp.float32) + b0[0], 0.0)
  h1[1] = jnp.maximum(
      jnp.dot(agg, w0[1], preferred_element_type=jnp.float32) + b0[1], 0.0)


@jax.jit
def _tc_layer1(acc0, x, deg3, w0s, b0s):
  return pl.pallas_call(
      _tc1_body,
      grid=(NBLKS,),
      in_specs=[pl.BlockSpec((NC, RBLK, D_IN), lambda i: (0, i, 0)),
                pl.BlockSpec((RBLK, D_IN), lambda i: (i, 0)),
                pl.BlockSpec((NC, RBLK, 1), lambda i: (0, i, 0)),
                pl.BlockSpec((NC, D_IN, D_IN), lambda i: (0, 0, 0)),
                pl.BlockSpec((NC, 1, D_IN), lambda i: (0, 0, 0))],
      out_specs=[pl.BlockSpec((NC, RBLK, D_IN), lambda i: (0, i, 0))],
      out_shape=[jax.ShapeDtypeStruct((NC, N_PAD, D_IN), jnp.float32)],
  )(acc0, x, deg3, w0s, b0s)[0]


def _tc2_body(y, h1, deg3, orig, al,
              w1, b1, wa1, ba1, wa2, ba2, out):
  deg = deg3[0] + deg3[1] + 1.0
  agga = (y[0] + h1[0]) / deg
  aggb = (y[1] + h1[1]) / deg
  h2 = jnp.dot(agga, w1[...][:D_IN],
               preferred_element_type=jnp.float32)
  h2 = h2 + jnp.dot(aggb, w1[...][D_IN:],
                    preferred_element_type=jnp.float32)
  h2 = jnp.maximum(h2 + b1[...], 0.0)
  a1 = jnp.maximum(
      jnp.dot(h2, wa1[...], preferred_element_type=jnp.float32) + ba1[...],
      0.0)
  adj = jnp.dot(a1, wa2[...], preferred_element_type=jnp.float32) + ba2[...]
  alpha = 1.0 / (1.0 + jnp.exp(-al[...]))
  out[...] = alpha * orig[...] + (1.0 - alpha) * adj


@jax.jit
def _tc_layer2(y, h1, deg3, orig, al, w1, b1, wa1, ba1, wa2, ba2):
  full = lambda r, c: pl.BlockSpec((r, c), lambda i: (0, 0))
  return pl.pallas_call(
      _tc2_body,
      grid=(NBLKS,),
      in_specs=[pl.BlockSpec((NC, RBLK, D_IN), lambda i: (0, i, 0)),
                pl.BlockSpec((NC, RBLK, D_IN), lambda i: (0, i, 0)),
                pl.BlockSpec((NC, RBLK, 1), lambda i: (0, i, 0)),
                pl.BlockSpec((RBLK, 1), lambda i: (i, 0)), full(1, 1),
                full(D_H, D_H), full(1, D_H),
                full(D_H, D_IN), full(1, D_IN),
                full(D_IN, 1), full(1, 1)],
      out_specs=[pl.BlockSpec((RBLK, 1), lambda i: (i, 0))],
      out_shape=[jax.ShapeDtypeStruct((N, 1), jnp.float32)],
  )(y, h1, deg3, orig, al, w1, b1, wa1, ba1, wa2, ba2)[0]


def kernel(x, edge_index, original_scores, W0, b0, W1, b1, Wa1, ba1, Wa2,
           ba2, alpha_logit):
  extra = jnp.arange(E_PAD - E, dtype=jnp.int32)
  src = jnp.concatenate([edge_index[0], extra % N])
  dst = jnp.concatenate([edge_index[1], N + extra % (N_PAD - N)])
  zrows = jnp.zeros((ROWS_PER_TILE, D_IN), jnp.float32)
  zvec = jnp.zeros((ROWS_PER_TILE,), jnp.float32)
  w0s = jnp.stack([W0[:, :D_IN], W0[:, D_IN:]])
  b0s = jnp.stack([b0[None, :D_IN], b0[None, D_IN:]])

  acc0, deg = _sc_agg_layer1(x, src, dst, zrows, zvec)
  deg3 = deg[:, :, None]

  h1 = _tc_layer1(acc0, x, deg3, w0s, b0s)
  y = _sc_agg_layer2(h1, src, dst, zrows)

  refined = _tc_layer2(y, h1, deg3, original_scores[:, None],
                       alpha_logit[None, None],
                       W1, b1[None, :], Wa1, ba1[None, :], Wa2,
                       ba2.reshape(1, 1))
  return refined[:, 0]
